# baseline (speedup 1.0000x reference)
import functools

import jax
import jax.numpy as jnp
from jax.experimental import pallas as pl


_ROW_BLOCK = 512


def _loss_kernel(xa_ref, ya_ref, xb_ref, yb_ref, out_ref, *, n, k):
    blk = min(_ROW_BLOCK, n)
    steps = n // blk
    ya = ya_ref[0].astype(jnp.bfloat16)
    yb = yb_ref[0].astype(jnp.bfloat16)

    def tile(x_ref, y, i):
        xb_blk = x_ref[0, i * blk:(i + 1) * blk, :].astype(jnp.bfloat16)
        return jax.lax.dot_general(
            xb_blk, y, (((1,), (1,)), ((), ())),
            preferred_element_type=jnp.float32,
        )

    acc2 = jnp.full((1, n), jnp.inf, dtype=jnp.float32)
    acc1 = jnp.full((1, n), jnp.inf, dtype=jnp.float32)
    for i in range(steps):
        t2 = tile(xa_ref, ya, i)
        t1 = tile(xb_ref, yb, i)
        acc2 = jnp.minimum(acc2, jnp.min(t2, axis=0, keepdims=True))
        acc1 = jnp.minimum(acc1, jnp.min(t1, axis=0, keepdims=True))

    d = jnp.sqrt(jnp.maximum(jnp.concatenate([acc1, acc2], axis=0), 0.0))
    loss_cd = jnp.sum(d) * jnp.float32(1.0 / n)

    bits = jax.lax.bitcast_convert_type(d, jnp.int32)

    def bs(_, lohi):
        lo, hi = lohi
        mid = lo + (hi - lo + 1) // 2
        cnt = jnp.sum((bits >= mid).astype(jnp.int32), axis=1, keepdims=True)
        take = cnt >= k
        return (jnp.where(take, mid, lo), jnp.where(take, hi, mid - 1))

    lo0 = jnp.zeros((2, 1), jnp.int32)
    hi0 = jnp.full((2, 1), 0x7F000000, jnp.int32)
    lo, _ = jax.lax.fori_loop(0, 32, bs, (lo0, hi0))
    v = jax.lax.bitcast_convert_type(lo, jnp.float32)
    sum_gt = jnp.sum(jnp.where(d > v, d, 0.0))
    cnt_gt = jnp.sum((d > v).astype(jnp.float32), axis=1, keepdims=True)
    corr = jnp.sum((jnp.float32(k) - cnt_gt) * v)
    loss_w = (sum_gt + corr) * jnp.float32(1.0 / k)
    out_ref[0, 0, :] = jnp.full((128,), loss_cd + 3.0 * loss_w, jnp.float32)


def _hi_lo(x2):
    hi_f = jax.lax.bitcast_convert_type(
        jax.lax.bitcast_convert_type(x2, jnp.int32) & jnp.int32(-65536),
        jnp.float32)
    return hi_f, x2 - hi_f


def _operands(x, y):
    b, n, _ = x.shape
    x2 = jnp.sum(x * x, axis=-1, keepdims=True)
    y2 = jnp.sum(y * y, axis=-1, keepdims=True)
    x2hi, x2lo = _hi_lo(x2)
    y2hi, y2lo = _hi_lo(y2)
    ones = jnp.ones((b, n, 1), jnp.float32)
    zpad = jnp.zeros((b, n, 1), jnp.float32)
    xa = jnp.concatenate([-2.0 * x, x2hi, x2lo, ones, ones, zpad], axis=-1)
    ya = jnp.concatenate([y, ones, ones, y2hi, y2lo, zpad], axis=-1)
    return xa, ya


def kernel(pred_pointclouds, gt_pointclouds):
    pred = pred_pointclouds.astype(jnp.float32)
    gt = gt_pointclouds.astype(jnp.float32)
    b, n, _ = pred.shape
    k = int(0.5 * n)

    xa, ya = _operands(pred, gt)
    xb, yb = _operands(gt, pred)

    spec = pl.BlockSpec((1, n, 8), lambda i: (i, 0, 0))
    losses = pl.pallas_call(
        functools.partial(_loss_kernel, n=n, k=k),
        grid=(b,),
        in_specs=[spec, spec, spec, spec],
        out_specs=pl.BlockSpec((1, 1, 128), lambda i: (i, 0, 0)),
        out_shape=jax.ShapeDtypeStruct((b, 1, 128), jnp.float32),
    )(xa, ya, xb, yb)
    return jnp.sum(losses[:, 0, 0]) / b


# speedup vs baseline: 2.1908x; 2.1908x over previous
import functools

import jax
import jax.numpy as jnp
from jax.experimental import pallas as pl


_ROW_BLOCK = 1024


def _min_over_rows(x_ref, y_ref, n):
    blk = min(_ROW_BLOCK, n)
    y_bf = y_ref[0].astype(jnp.bfloat16)

    def step(i, acc):
        xb = x_ref[0, pl.ds(i * blk, blk), :]
        xy = jax.lax.dot_general(
            xb.astype(jnp.bfloat16), y_bf, (((1,), (1,)), ((), ())),
            preferred_element_type=jnp.float32,
        )
        x2 = jnp.sum(xb * xb, axis=1, keepdims=True)
        t = x2 - 2.0 * xy
        return jnp.minimum(acc, jnp.min(t, axis=0, keepdims=True))

    acc0 = jnp.full((1, n), jnp.inf, dtype=jnp.float32)
    return jax.lax.fori_loop(0, n // blk, step, acc0)


def _loss_kernel(p_ref, g_ref, norms_ref, out_ref, *, n, k):
    p2 = norms_ref[0, 0:1, :]
    g2 = norms_ref[0, 1:2, :]
    m2 = _min_over_rows(p_ref, g_ref, n)
    m1 = _min_over_rows(g_ref, p_ref, n)
    m = jnp.concatenate([m1 + p2, m2 + g2], axis=0)
    d = jnp.sqrt(jnp.maximum(m, 0.0))
    loss_cd = jnp.sum(d) * jnp.float32(1.0 / n)

    bits = jax.lax.bitcast_convert_type(d, jnp.int32)

    def bs(_, lohi):
        lo, hi = lohi
        mid = lo + (hi - lo + 1) // 2
        cnt = jnp.sum((bits >= mid).astype(jnp.int32), axis=1, keepdims=True)
        take = cnt >= k
        return (jnp.where(take, mid, lo), jnp.where(take, hi, mid - 1))

    lo0 = jnp.zeros((2, 1), jnp.int32)
    hi0 = jnp.full((2, 1), 0x7F000000, jnp.int32)
    lo, _ = jax.lax.fori_loop(0, 32, bs, (lo0, hi0))
    v = jax.lax.bitcast_convert_type(lo, jnp.float32)
    sum_gt = jnp.sum(jnp.where(d > v, d, 0.0))
    cnt_gt = jnp.sum((d > v).astype(jnp.float32), axis=1, keepdims=True)
    corr = jnp.sum((jnp.float32(k) - cnt_gt) * v)
    loss_w = (sum_gt + corr) * jnp.float32(1.0 / k)
    out_ref[0, 0, :] = jnp.full((128,), loss_cd + 3.0 * loss_w, jnp.float32)


def kernel(pred_pointclouds, gt_pointclouds):
    pred = pred_pointclouds.astype(jnp.float32)
    gt = gt_pointclouds.astype(jnp.float32)
    b, n, _ = pred.shape
    k = int(0.5 * n)

    zpad = jnp.zeros((b, n, 5), jnp.float32)
    p_pad = jnp.concatenate([pred, zpad], axis=-1)
    g_pad = jnp.concatenate([gt, zpad], axis=-1)
    p2 = jnp.sum(pred * pred, axis=-1)
    g2 = jnp.sum(gt * gt, axis=-1)
    norms = jnp.concatenate(
        [p2[:, None, :], g2[:, None, :],
         jnp.zeros((b, 6, n), jnp.float32)], axis=1)

    spec = pl.BlockSpec((1, n, 8), lambda i: (i, 0, 0))
    losses = pl.pallas_call(
        functools.partial(_loss_kernel, n=n, k=k),
        grid=(b,),
        in_specs=[spec, spec, pl.BlockSpec((1, 8, n), lambda i: (i, 0, 0))],
        out_specs=pl.BlockSpec((1, 1, 128), lambda i: (i, 0, 0)),
        out_shape=jax.ShapeDtypeStruct((b, 1, 128), jnp.float32),
    )(p_pad, g_pad, norms)
    return jnp.sum(losses[:, 0, 0]) / b


# speedup vs baseline: 2.2462x; 1.0253x over previous
import functools

import jax
import jax.numpy as jnp
from jax.experimental import pallas as pl


_ROW_BLOCK = 2048


def _min_over_rows(x_ref, y_ref, n):
    blk = min(_ROW_BLOCK, n)
    y_bf = y_ref[0].astype(jnp.bfloat16)

    def step(i, acc):
        xb = x_ref[0, pl.ds(i * blk, blk), :]
        xy = jax.lax.dot_general(
            xb.astype(jnp.bfloat16), y_bf, (((1,), (1,)), ((), ())),
            preferred_element_type=jnp.float32,
        )
        x2 = jnp.sum(xb * xb, axis=1, keepdims=True)
        t = x2 - 2.0 * xy
        return jnp.minimum(acc, jnp.min(t, axis=0, keepdims=True))

    acc0 = jnp.full((1, n), jnp.inf, dtype=jnp.float32)
    return jax.lax.fori_loop(0, n // blk, step, acc0)


def _loss_kernel(p_ref, g_ref, norms_ref, out_ref, *, n, k):
    p2 = norms_ref[0, 0:1, :]
    g2 = norms_ref[0, 1:2, :]
    m2 = _min_over_rows(p_ref, g_ref, n)
    m1 = _min_over_rows(g_ref, p_ref, n)
    m = jnp.concatenate([m1 + p2, m2 + g2], axis=0)
    d = jnp.sqrt(jnp.maximum(m, 0.0))
    loss_cd = jnp.sum(d) * jnp.float32(1.0 / n)

    bits = jax.lax.bitcast_convert_type(d, jnp.int32)

    def bs(_, lohi):
        lo, hi = lohi
        mid = lo + (hi - lo + 1) // 2
        cnt = jnp.sum((bits >= mid).astype(jnp.int32), axis=1, keepdims=True)
        take = cnt >= k
        return (jnp.where(take, mid, lo), jnp.where(take, hi, mid - 1))

    lo0 = jnp.zeros((2, 1), jnp.int32)
    hi0 = jnp.full((2, 1), 0x7F000000, jnp.int32)
    lo, _ = jax.lax.fori_loop(0, 32, bs, (lo0, hi0))
    v = jax.lax.bitcast_convert_type(lo, jnp.float32)
    sum_gt = jnp.sum(jnp.where(d > v, d, 0.0))
    cnt_gt = jnp.sum((d > v).astype(jnp.float32), axis=1, keepdims=True)
    corr = jnp.sum((jnp.float32(k) - cnt_gt) * v)
    loss_w = (sum_gt + corr) * jnp.float32(1.0 / k)
    out_ref[0, 0, :] = jnp.full((128,), loss_cd + 3.0 * loss_w, jnp.float32)


def kernel(pred_pointclouds, gt_pointclouds):
    pred = pred_pointclouds.astype(jnp.float32)
    gt = gt_pointclouds.astype(jnp.float32)
    b, n, _ = pred.shape
    k = int(0.5 * n)

    zpad = jnp.zeros((b, n, 5), jnp.float32)
    p_pad = jnp.concatenate([pred, zpad], axis=-1)
    g_pad = jnp.concatenate([gt, zpad], axis=-1)
    p2 = jnp.sum(pred * pred, axis=-1)
    g2 = jnp.sum(gt * gt, axis=-1)
    norms = jnp.concatenate(
        [p2[:, None, :], g2[:, None, :],
         jnp.zeros((b, 6, n), jnp.float32)], axis=1)

    spec = pl.BlockSpec((1, n, 8), lambda i: (i, 0, 0))
    losses = pl.pallas_call(
        functools.partial(_loss_kernel, n=n, k=k),
        grid=(b,),
        in_specs=[spec, spec, pl.BlockSpec((1, 8, n), lambda i: (i, 0, 0))],
        out_specs=pl.BlockSpec((1, 1, 128), lambda i: (i, 0, 0)),
        out_shape=jax.ShapeDtypeStruct((b, 1, 128), jnp.float32),
    )(p_pad, g_pad, norms)
    return jnp.sum(losses[:, 0, 0]) / b
